# speedup vs baseline: 4.5445x; 4.5445x over previous
import jax
import jax.numpy as jnp
from jax.experimental import pallas as pl
from jax.experimental.pallas import tpu as pltpu

_S = 4096
_R = 256
_G = _S // _R


def _chamfer_body(xc_ref, yr_ref, out_ref, colmin_ref, rowacc_ref):
    i = pl.program_id(0)

    @pl.when(i == 0)
    def _init():
        colmin_ref[...] = jnp.full((1, _S), jnp.inf, dtype=jnp.float32)
        rowacc_ref[...] = jnp.zeros((1, 1), dtype=jnp.float32)

    xs = xc_ref[...]
    dx = xs[:, 0:1] - yr_ref[0:1, :]
    dy = xs[:, 1:2] - yr_ref[1:2, :]
    dz = xs[:, 2:3] - yr_ref[2:3, :]
    d2 = dx * dx + dy * dy + dz * dz

    rowmin = jnp.min(d2, axis=1)
    rowacc_ref[...] = rowacc_ref[...] + jnp.sum(jnp.sqrt(rowmin))
    colmin_ref[...] = jnp.minimum(colmin_ref[...],
                                  jnp.min(d2, axis=0, keepdims=True))

    @pl.when(i == _G - 1)
    def _fin():
        loss2 = jnp.sum(jnp.sqrt(colmin_ref[...])) / _S
        out_ref[...] = rowacc_ref[...] / _S + loss2


def kernel(X, Y):
    Xc = X[0]
    Yr = jnp.transpose(Y[0], (1, 0))
    out = pl.pallas_call(
        _chamfer_body,
        grid=(_G,),
        in_specs=[
            pl.BlockSpec((_R, 3), lambda i: (i, 0)),
            pl.BlockSpec((3, _S), lambda i: (0, 0)),
        ],
        out_specs=pl.BlockSpec((1, 1), lambda i: (0, 0)),
        out_shape=jax.ShapeDtypeStruct((1, 1), jnp.float32),
        scratch_shapes=[
            pltpu.VMEM((1, _S), jnp.float32),
            pltpu.VMEM((1, 1), jnp.float32),
        ],
    )(Xc, Yr)
    return out[0, 0]
